# baseline (speedup 1.0000x reference)
import functools

import jax
import jax.numpy as jnp
from jax import lax
from jax.experimental import pallas as pl
from jax.experimental.pallas import tpu as pltpu
from jax.experimental.pallas import tpu_sc as plsc

EMB = 128
NUM_ITEMS = 16384
MAX_SIZE = 20

NC = 2
NS = 16
NW = NC * NS
C = 16
ROWS = C * MAX_SIZE
CPW = NUM_ITEMS // (NW * C)
IPW = NUM_ITEMS // NW
NSPLIT = 4
GROWS = MAX_SIZE // NSPLIT
GLEN = ROWS // NSPLIT
NVREG = EMB // 16


def _vlane_gather(x, idx):
    dnums = lax.GatherDimensionNumbers(
        offset_dims=(), collapsed_slice_dims=(0,), start_index_map=(0,))
    return lax.gather(x, idx[:, None], dnums, slice_sizes=(1,),
                      mode=lax.GatherScatterMode.PROMISE_IN_BOUNDS)


def _sc_body(idxw_hbm, len_hbm, table_hbm, out_hbm,
             len_all, perm_a, perm_b, tm_a, tm_b, rows_a, rows_b,
             row0_v, out_a, out_b,
             sem_ia, sem_ib, sem_ga, sem_gb, sem_oa, sem_ob):
    wid = lax.axis_index("s") * NC + lax.axis_index("c")
    chunk0 = wid * CPW

    pltpu.sync_copy(table_hbm.at[pl.ds(0, 1)], row0_v)
    pltpu.sync_copy(len_hbm.at[pl.ds(wid * IPW, IPW)], len_all)

    iota20 = lax.iota(jnp.int32, 16) * MAX_SIZE

    def build_perm(ci, perm_buf):
        base = (chunk0 + ci) * ROWS + iota20
        for j in range(MAX_SIZE):
            perm_buf[pl.ds(j * 16, 16)] = base + j

    def issue_idx(perm_buf, tm_buf, sem):
        for k in range(NSPLIT):
            pltpu.async_copy(
                idxw_hbm.at[perm_buf.at[pl.ds(k * GLEN, GLEN)]],
                tm_buf.at[pl.ds(k * GLEN, GLEN)], sem)

    def drain_idx(perm_buf, tm_buf, sem):
        for k in range(NSPLIT):
            pltpu.make_async_copy(
                idxw_hbm.at[perm_buf.at[pl.ds(k * GLEN, GLEN)]],
                tm_buf.at[pl.ds(k * GLEN, GLEN)], sem).wait()

    def issue_rows(tm_buf, rows_buf, sem):
        for k in range(NSPLIT):
            pltpu.async_copy(
                table_hbm.at[tm_buf.at[pl.ds(k * GLEN, GLEN)]],
                rows_buf.at[pl.ds(k * GLEN, GLEN)], sem)

    def drain_rows(tm_buf, rows_buf, sem):
        for k in range(NSPLIT):
            pltpu.make_async_copy(
                table_hbm.at[tm_buf.at[pl.ds(k * GLEN, GLEN)]],
                rows_buf.at[pl.ds(k * GLEN, GLEN)], sem).wait()

    def prep(ci, tm_buf):
        zc = jnp.zeros((16,), jnp.float32)
        for j in range(MAX_SIZE):
            tok = tm_buf[pl.ds(j * 16, 16)]
            zc = zc + jnp.where(tok == 0, jnp.float32(1.0), jnp.float32(0.0))
        rcpv = jnp.float32(1.0) / len_all[pl.ds(ci * C, C)].astype(jnp.float32)
        return rcpv, zc * rcpv

    def compute(rows_buf, out_buf, rcpv, zrv):
        def item_body(i, c2):
            bidx = jnp.full((16,), i, jnp.int32)
            a = _vlane_gather(rcpv, bidx)
            b = _vlane_gather(zrv, bidx)
            for v in range(NVREG):
                sl = pl.ds(v * 16, 16)
                acc = rows_buf[i, sl]
                for j in range(1, MAX_SIZE):
                    acc = acc + rows_buf[j * 16 + i, sl]
                out_buf[i, sl] = acc * a - b * row0_v[0, sl]
            return c2
        lax.fori_loop(0, C, item_body, 0, unroll=False)

    def store(ci, out_buf, sem):
        pltpu.async_copy(out_buf, out_hbm.at[pl.ds((chunk0 + ci) * C, C)], sem)

    def drain_store(out_buf, sem):
        pltpu.make_async_copy(out_buf, out_hbm.at[pl.ds(0, C)], sem).wait()

    build_perm(0, perm_a)
    issue_idx(perm_a, tm_a, sem_ia)
    drain_idx(perm_a, tm_a, sem_ia)
    issue_rows(tm_a, rows_a, sem_ga)
    build_perm(1, perm_b)
    issue_idx(perm_b, tm_b, sem_ib)

    def pair_body(p, carry):
        ca = 2 * p
        cb = 2 * p + 1
        drain_idx(perm_b, tm_b, sem_ib)
        issue_rows(tm_b, rows_b, sem_gb)
        build_perm(jnp.minimum(ca + 2, CPW - 1), perm_a)
        rcp_a, zr_a = prep(ca, tm_a)
        drain_rows(tm_a, rows_a, sem_ga)

        @pl.when(p > 0)
        def _():
            drain_store(out_a, sem_oa)

        compute(rows_a, out_a, rcp_a, zr_a)
        store(ca, out_a, sem_oa)
        issue_idx(perm_a, tm_a, sem_ia)

        drain_idx(perm_a, tm_a, sem_ia)
        issue_rows(tm_a, rows_a, sem_ga)
        build_perm(jnp.minimum(cb + 2, CPW - 1), perm_b)
        rcp_b, zr_b = prep(cb, tm_b)
        drain_rows(tm_b, rows_b, sem_gb)

        @pl.when(p > 0)
        def _():
            drain_store(out_b, sem_ob)

        compute(rows_b, out_b, rcp_b, zr_b)
        store(cb, out_b, sem_ob)
        issue_idx(perm_b, tm_b, sem_ib)
        return carry

    lax.fori_loop(0, CPW // 2, pair_body, 0, unroll=False)
    drain_idx(perm_b, tm_b, sem_ib)
    drain_rows(tm_a, rows_a, sem_ga)
    drain_store(out_a, sem_oa)
    drain_store(out_b, sem_ob)


def kernel(input_tensor, item_size, emb_table):
    idx_w = input_tensor.reshape(NUM_ITEMS * MAX_SIZE).astype(jnp.int32)
    lens = item_size.astype(jnp.int32)

    mesh = plsc.VectorSubcoreMesh(core_axis_name="c", subcore_axis_name="s")
    run = functools.partial(
        pl.kernel,
        mesh=mesh,
        out_type=jax.ShapeDtypeStruct((NUM_ITEMS, EMB), jnp.float32),
        scratch_types=[
            pltpu.VMEM((IPW,), jnp.int32),
            pltpu.VMEM((ROWS,), jnp.int32),
            pltpu.VMEM((ROWS,), jnp.int32),
            pltpu.VMEM((ROWS,), jnp.int32),
            pltpu.VMEM((ROWS,), jnp.int32),
            pltpu.VMEM((ROWS, EMB), jnp.float32),
            pltpu.VMEM((ROWS, EMB), jnp.float32),
            pltpu.VMEM((1, EMB), jnp.float32),
            pltpu.VMEM((C, EMB), jnp.float32),
            pltpu.VMEM((C, EMB), jnp.float32),
            pltpu.SemaphoreType.DMA,
            pltpu.SemaphoreType.DMA,
            pltpu.SemaphoreType.DMA,
            pltpu.SemaphoreType.DMA,
            pltpu.SemaphoreType.DMA,
            pltpu.SemaphoreType.DMA,
        ],
    )(_sc_body)
    return run(idx_w, lens, emb_table)


# speedup vs baseline: 1.0731x; 1.0731x over previous
import functools

import jax
import jax.numpy as jnp
from jax import lax
from jax.experimental import pallas as pl
from jax.experimental.pallas import tpu as pltpu
from jax.experimental.pallas import tpu_sc as plsc

EMB = 128
NUM_ITEMS = 16384
MAX_SIZE = 20

NC = 2
NS = 16
NW = NC * NS
C = 16
ROWS = C * MAX_SIZE
CPW = NUM_ITEMS // (NW * C)
IPW = NUM_ITEMS // NW
NSPLIT = 4
GROWS = MAX_SIZE // NSPLIT
GLEN = ROWS // NSPLIT
NVREG = EMB // 16


def _vlane_gather(x, idx):
    dnums = lax.GatherDimensionNumbers(
        offset_dims=(), collapsed_slice_dims=(0,), start_index_map=(0,))
    return lax.gather(x, idx[:, None], dnums, slice_sizes=(1,),
                      mode=lax.GatherScatterMode.PROMISE_IN_BOUNDS)


def _sc_body(idxw_hbm, len_hbm, table_hbm, out_hbm,
             len_all, perm_a, perm_b, tm_a, tm_b, rows_a, rows_b,
             row0_v, out_a, out_b,
             sem_ia, sem_ib, sem_ga, sem_gb, sem_oa, sem_ob):
    wid = lax.axis_index("s") * NC + lax.axis_index("c")
    chunk0 = wid * CPW

    pltpu.sync_copy(table_hbm.at[pl.ds(0, 1)], row0_v)
    pltpu.sync_copy(len_hbm.at[pl.ds(wid * IPW, IPW)], len_all)

    iota20 = lax.iota(jnp.int32, 16) * MAX_SIZE

    def build_perm(ci, perm_buf):
        base = (chunk0 + ci) * ROWS + iota20
        for j in range(MAX_SIZE):
            perm_buf[pl.ds(j * 16, 16)] = base + j

    def issue_idx(perm_buf, tm_buf, sem):
        for k in range(NSPLIT):
            pltpu.async_copy(
                idxw_hbm.at[perm_buf.at[pl.ds(k * GLEN, GLEN)]],
                tm_buf.at[pl.ds(k * GLEN, GLEN)], sem)

    def drain_idx(perm_buf, tm_buf, sem):
        for k in range(NSPLIT):
            pltpu.make_async_copy(
                idxw_hbm.at[perm_buf.at[pl.ds(k * GLEN, GLEN)]],
                tm_buf.at[pl.ds(k * GLEN, GLEN)], sem).wait()

    def issue_rows(tm_buf, rows_buf, sem):
        for k in range(NSPLIT):
            pltpu.async_copy(
                table_hbm.at[tm_buf.at[pl.ds(k * GLEN, GLEN)]],
                rows_buf.at[pl.ds(k * GLEN, GLEN)], sem)

    def drain_rows(tm_buf, rows_buf, sem):
        for k in range(NSPLIT):
            pltpu.make_async_copy(
                table_hbm.at[tm_buf.at[pl.ds(k * GLEN, GLEN)]],
                rows_buf.at[pl.ds(k * GLEN, GLEN)], sem).wait()

    def prep(ci, tm_buf):
        zc = jnp.zeros((16,), jnp.float32)
        for j in range(MAX_SIZE):
            tok = tm_buf[pl.ds(j * 16, 16)]
            zc = zc + jnp.where(tok == 0, jnp.float32(1.0), jnp.float32(0.0))
        rcpv = jnp.float32(1.0) / len_all[pl.ds(ci * C, C)].astype(jnp.float32)
        return rcpv, zc * rcpv

    def compute(rows_buf, out_buf, rcpv, zrv):
        row0 = [row0_v[0, pl.ds(v * 16, 16)] for v in range(NVREG)]

        def item_body(i, c2):
            bidx = jnp.full((16,), i, jnp.int32)
            a = _vlane_gather(rcpv, bidx)
            b = _vlane_gather(zrv, bidx)
            acc = [rows_buf[i, pl.ds(v * 16, 16)] for v in range(NVREG)]
            for j in range(1, MAX_SIZE):
                for v in range(NVREG):
                    acc[v] = acc[v] + rows_buf[j * 16 + i, pl.ds(v * 16, 16)]
            for v in range(NVREG):
                out_buf[i, pl.ds(v * 16, 16)] = acc[v] * a - b * row0[v]
            return c2
        lax.fori_loop(0, C, item_body, 0, unroll=False)

    def store(ci, out_buf, sem):
        pltpu.async_copy(out_buf, out_hbm.at[pl.ds((chunk0 + ci) * C, C)], sem)

    def drain_store(out_buf, sem):
        pltpu.make_async_copy(out_buf, out_hbm.at[pl.ds(0, C)], sem).wait()

    build_perm(0, perm_a)
    issue_idx(perm_a, tm_a, sem_ia)
    drain_idx(perm_a, tm_a, sem_ia)
    issue_rows(tm_a, rows_a, sem_ga)
    build_perm(1, perm_b)
    issue_idx(perm_b, tm_b, sem_ib)

    def pair_body(p, carry):
        ca = 2 * p
        cb = 2 * p + 1
        drain_idx(perm_b, tm_b, sem_ib)
        issue_rows(tm_b, rows_b, sem_gb)
        build_perm(jnp.minimum(ca + 2, CPW - 1), perm_a)
        rcp_a, zr_a = prep(ca, tm_a)
        drain_rows(tm_a, rows_a, sem_ga)

        @pl.when(p > 0)
        def _():
            drain_store(out_a, sem_oa)

        compute(rows_a, out_a, rcp_a, zr_a)
        store(ca, out_a, sem_oa)
        issue_idx(perm_a, tm_a, sem_ia)

        drain_idx(perm_a, tm_a, sem_ia)
        issue_rows(tm_a, rows_a, sem_ga)
        build_perm(jnp.minimum(cb + 2, CPW - 1), perm_b)
        rcp_b, zr_b = prep(cb, tm_b)
        drain_rows(tm_b, rows_b, sem_gb)

        @pl.when(p > 0)
        def _():
            drain_store(out_b, sem_ob)

        compute(rows_b, out_b, rcp_b, zr_b)
        store(cb, out_b, sem_ob)
        issue_idx(perm_b, tm_b, sem_ib)
        return carry

    lax.fori_loop(0, CPW // 2, pair_body, 0, unroll=False)
    drain_idx(perm_b, tm_b, sem_ib)
    drain_rows(tm_a, rows_a, sem_ga)
    drain_store(out_a, sem_oa)
    drain_store(out_b, sem_ob)


def kernel(input_tensor, item_size, emb_table):
    idx_w = input_tensor.reshape(NUM_ITEMS * MAX_SIZE).astype(jnp.int32)
    lens = item_size.astype(jnp.int32)

    mesh = plsc.VectorSubcoreMesh(core_axis_name="c", subcore_axis_name="s")
    run = functools.partial(
        pl.kernel,
        mesh=mesh,
        out_type=jax.ShapeDtypeStruct((NUM_ITEMS, EMB), jnp.float32),
        scratch_types=[
            pltpu.VMEM((IPW,), jnp.int32),
            pltpu.VMEM((ROWS,), jnp.int32),
            pltpu.VMEM((ROWS,), jnp.int32),
            pltpu.VMEM((ROWS,), jnp.int32),
            pltpu.VMEM((ROWS,), jnp.int32),
            pltpu.VMEM((ROWS, EMB), jnp.float32),
            pltpu.VMEM((ROWS, EMB), jnp.float32),
            pltpu.VMEM((1, EMB), jnp.float32),
            pltpu.VMEM((C, EMB), jnp.float32),
            pltpu.VMEM((C, EMB), jnp.float32),
            pltpu.SemaphoreType.DMA,
            pltpu.SemaphoreType.DMA,
            pltpu.SemaphoreType.DMA,
            pltpu.SemaphoreType.DMA,
            pltpu.SemaphoreType.DMA,
            pltpu.SemaphoreType.DMA,
        ],
    )(_sc_body)
    return run(idx_w, lens, emb_table)


# speedup vs baseline: 1.2412x; 1.1566x over previous
import functools

import jax
import jax.numpy as jnp
from jax import lax
from jax.experimental import pallas as pl
from jax.experimental.pallas import tpu as pltpu
from jax.experimental.pallas import tpu_sc as plsc

EMB = 128
NUM_ITEMS = 16384
MAX_SIZE = 20

NC = 2
NS = 16
NW = NC * NS
C = 16
ROWS = C * MAX_SIZE
CPW = NUM_ITEMS // (NW * C)
IPW = NUM_ITEMS // NW
NSPLIT = 4
GLEN = ROWS // NSPLIT
NVREG = EMB // 16
D = 4


def _vlane_gather(x, idx):
    dnums = lax.GatherDimensionNumbers(
        offset_dims=(), collapsed_slice_dims=(0,), start_index_map=(0,))
    return lax.gather(x, idx[:, None], dnums, slice_sizes=(1,),
                      mode=lax.GatherScatterMode.PROMISE_IN_BOUNDS)


def _sc_body(idxw_hbm, len_hbm, table_hbm, out_hbm,
             len_all, row0_v, *rest):
    perm = rest[0:D]
    tm = rest[D:2 * D]
    acc = rest[2 * D:3 * D]
    outb = rest[3 * D:4 * D]
    sem_i = rest[4 * D:5 * D]
    sem_r = rest[5 * D:6 * D]
    sem_o = rest[6 * D:7 * D]
    wid = lax.axis_index("s") * NC + lax.axis_index("c")
    chunk0 = wid * CPW

    pltpu.sync_copy(table_hbm.at[pl.ds(0, 1)], row0_v)
    pltpu.sync_copy(len_hbm.at[pl.ds(wid * IPW, IPW)], len_all)

    iota20 = lax.iota(jnp.int32, 16) * MAX_SIZE
    zeros16 = jnp.zeros((16,), jnp.float32)
    row0 = [row0_v[0, pl.ds(v * 16, 16)] for v in range(NVREG)]

    def build_perm(ci, k):
        base = (chunk0 + ci) * ROWS + iota20
        for j in range(MAX_SIZE):
            perm[k][pl.ds(j * 16, 16)] = base + j

    def issue_idx(k):
        for s in range(NSPLIT):
            pltpu.async_copy(
                idxw_hbm.at[perm[k].at[pl.ds(s * GLEN, GLEN)]],
                tm[k].at[pl.ds(s * GLEN, GLEN)], sem_i[k])

    def drain_idx(k):
        for s in range(NSPLIT):
            pltpu.make_async_copy(
                idxw_hbm.at[perm[k].at[pl.ds(s * GLEN, GLEN)]],
                tm[k].at[pl.ds(s * GLEN, GLEN)], sem_i[k]).wait()

    def issue_rows(k):
        for j in range(MAX_SIZE):
            pltpu.async_copy(
                table_hbm.at[tm[k].at[pl.ds(j * 16, 16)]],
                acc[k], sem_r[k], add=True)

    def drain_rows(k):
        for j in range(MAX_SIZE):
            pltpu.make_async_copy(
                table_hbm.at[tm[k].at[pl.ds(j * 16, 16)]],
                acc[k], sem_r[k]).wait()

    def zero_acc(k):
        for i in range(C):
            for v in range(NVREG):
                acc[k][i, pl.ds(v * 16, 16)] = zeros16

    def prep(ci, k):
        zc = jnp.zeros((16,), jnp.float32)
        for j in range(MAX_SIZE):
            tok = tm[k][pl.ds(j * 16, 16)]
            zc = zc + jnp.where(tok == 0, jnp.float32(1.0), jnp.float32(0.0))
        rcpv = jnp.float32(1.0) / len_all[pl.ds(ci * C, C)].astype(jnp.float32)
        return rcpv, zc * rcpv

    def scale(k, rcpv, zrv):
        def item_body(i, c2):
            bidx = jnp.full((16,), i, jnp.int32)
            a = _vlane_gather(rcpv, bidx)
            b = _vlane_gather(zrv, bidx)
            for v in range(NVREG):
                sl = pl.ds(v * 16, 16)
                outb[k][i, sl] = acc[k][i, sl] * a - b * row0[v]
            return c2
        lax.fori_loop(0, C, item_body, 0, unroll=False)

    def issue_out(ci, k):
        pltpu.async_copy(
            outb[k], out_hbm.at[pl.ds((chunk0 + ci) * C, C)], sem_o[k])

    def drain_out(k):
        pltpu.make_async_copy(
            outb[k], out_hbm.at[pl.ds(0, C)], sem_o[k]).wait()

    for k in range(D):
        zero_acc(k)
    build_perm(0, 0)
    issue_idx(0)
    build_perm(1, 1)
    issue_idx(1)
    drain_idx(0)
    issue_rows(0)
    build_perm(2, 2)
    issue_idx(2)
    drain_idx(1)
    issue_rows(1)
    build_perm(3, 3)
    issue_idx(3)

    def group_body(p, carry):
        for k in range(D):
            c = D * p + k
            drain_rows(k)
            rcpv, zrv = prep(c, k)

            @pl.when(p > 0)
            def _():
                drain_out(k)

            scale(k, rcpv, zrv)
            issue_out(c, k)
            zero_acc(k)

            @pl.when(c + 2 < CPW)
            def _():
                drain_idx((k + 2) % D)
                issue_rows((k + 2) % D)

            @pl.when(c + D < CPW)
            def _():
                build_perm(c + D, k)
                issue_idx(k)
        return carry

    lax.fori_loop(0, CPW // D, group_body, 0, unroll=False)
    for k in range(D):
        drain_out(k)


def kernel(input_tensor, item_size, emb_table):
    idx_w = input_tensor.reshape(NUM_ITEMS * MAX_SIZE).astype(jnp.int32)
    lens = item_size.astype(jnp.int32)

    mesh = plsc.VectorSubcoreMesh(core_axis_name="c", subcore_axis_name="s")
    run = functools.partial(
        pl.kernel,
        mesh=mesh,
        out_type=jax.ShapeDtypeStruct((NUM_ITEMS, EMB), jnp.float32),
        scratch_types=(
            [pltpu.VMEM((IPW,), jnp.int32),
             pltpu.VMEM((1, EMB), jnp.float32)]
            + [pltpu.VMEM((ROWS,), jnp.int32) for _ in range(D)]
            + [pltpu.VMEM((ROWS,), jnp.int32) for _ in range(D)]
            + [pltpu.VMEM((C, EMB), jnp.float32) for _ in range(D)]
            + [pltpu.VMEM((C, EMB), jnp.float32) for _ in range(D)]
            + [pltpu.SemaphoreType.DMA for _ in range(3 * D)]
        ),
    )(_sc_body)
    return run(idx_w, lens, emb_table)


# speedup vs baseline: 1.2910x; 1.0401x over previous
import functools

import jax
import jax.numpy as jnp
from jax import lax
from jax.experimental import pallas as pl
from jax.experimental.pallas import tpu as pltpu
from jax.experimental.pallas import tpu_sc as plsc

EMB = 128
NUM_ITEMS = 16384
MAX_SIZE = 20

NC = 2
NS = 16
NW = NC * NS
C = 16
ROWS = C * MAX_SIZE
CPW = NUM_ITEMS // (NW * C)
IPW = NUM_ITEMS // NW
NSPLIT = 4
GLEN = ROWS // NSPLIT
NVREG = EMB // 16
D = 8
RW = 4


def _vlane_gather(x, idx):
    dnums = lax.GatherDimensionNumbers(
        offset_dims=(), collapsed_slice_dims=(0,), start_index_map=(0,))
    return lax.gather(x, idx[:, None], dnums, slice_sizes=(1,),
                      mode=lax.GatherScatterMode.PROMISE_IN_BOUNDS)


def _sc_body(idxw_hbm, len_hbm, table_hbm, out_hbm,
             len_all, row0_v, *rest):
    perm = rest[0:D]
    tm = rest[D:2 * D]
    acc = rest[2 * D:3 * D]
    outb = rest[3 * D:4 * D]
    sem_i = rest[4 * D:5 * D]
    sem_r = rest[5 * D:6 * D]
    sem_o = rest[6 * D:7 * D]
    wid = lax.axis_index("s") * NC + lax.axis_index("c")
    chunk0 = wid * CPW

    pltpu.sync_copy(table_hbm.at[pl.ds(0, 1)], row0_v)
    pltpu.sync_copy(len_hbm.at[pl.ds(wid * IPW, IPW)], len_all)

    iota20 = lax.iota(jnp.int32, 16) * MAX_SIZE
    zeros16 = jnp.zeros((16,), jnp.float32)
    row0 = [row0_v[0, pl.ds(v * 16, 16)] for v in range(NVREG)]

    def build_perm(ci, k):
        base = (chunk0 + ci) * ROWS + iota20
        for j in range(MAX_SIZE):
            perm[k][pl.ds(j * 16, 16)] = base + j

    def issue_idx(k):
        for s in range(NSPLIT):
            pltpu.async_copy(
                idxw_hbm.at[perm[k].at[pl.ds(s * GLEN, GLEN)]],
                tm[k].at[pl.ds(s * GLEN, GLEN)], sem_i[k])

    def drain_idx(k):
        for s in range(NSPLIT):
            pltpu.make_async_copy(
                idxw_hbm.at[perm[k].at[pl.ds(s * GLEN, GLEN)]],
                tm[k].at[pl.ds(s * GLEN, GLEN)], sem_i[k]).wait()

    def issue_rows(k):
        for j in range(MAX_SIZE):
            pltpu.async_copy(
                table_hbm.at[tm[k].at[pl.ds(j * 16, 16)]],
                acc[k], sem_r[k], add=True)

    def drain_rows(k):
        for j in range(MAX_SIZE):
            pltpu.make_async_copy(
                table_hbm.at[tm[k].at[pl.ds(j * 16, 16)]],
                acc[k], sem_r[k]).wait()

    def zero_acc(k):
        for i in range(C):
            for v in range(NVREG):
                acc[k][i, pl.ds(v * 16, 16)] = zeros16

    def prep(ci, k):
        zc = jnp.zeros((16,), jnp.float32)
        for j in range(MAX_SIZE):
            tok = tm[k][pl.ds(j * 16, 16)]
            zc = zc + jnp.where(tok == 0, jnp.float32(1.0), jnp.float32(0.0))
        rcpv = jnp.float32(1.0) / len_all[pl.ds(ci * C, C)].astype(jnp.float32)
        return rcpv, zc * rcpv

    def scale(k, rcpv, zrv):
        def item_body(i, c2):
            bidx = jnp.full((16,), i, jnp.int32)
            a = _vlane_gather(rcpv, bidx)
            b = _vlane_gather(zrv, bidx)
            for v in range(NVREG):
                sl = pl.ds(v * 16, 16)
                outb[k][i, sl] = acc[k][i, sl] * a - b * row0[v]
            return c2
        lax.fori_loop(0, C, item_body, 0, unroll=False)

    def issue_out(ci, k):
        pltpu.async_copy(
            outb[k], out_hbm.at[pl.ds((chunk0 + ci) * C, C)], sem_o[k])

    def drain_out(k):
        pltpu.make_async_copy(
            outb[k], out_hbm.at[pl.ds(0, C)], sem_o[k]).wait()

    for k in range(D):
        zero_acc(k)
    for q in range(RW):
        build_perm(q, q)
        issue_idx(q)
    for q in range(RW):
        drain_idx(q)
        issue_rows(q)
        if RW + q < D:
            build_perm(RW + q, RW + q)
            issue_idx(RW + q)

    def group_body(p, carry):
        for k in range(D):
            c = D * p + k
            drain_rows(k)
            rcpv, zrv = prep(c, k)

            @pl.when(p > 0)
            def _():
                drain_out(k)

            scale(k, rcpv, zrv)
            issue_out(c, k)
            zero_acc(k)

            @pl.when(c + RW < CPW)
            def _():
                drain_idx((k + RW) % D)
                issue_rows((k + RW) % D)

            @pl.when(c + D < CPW)
            def _():
                build_perm(c + D, k)
                issue_idx(k)
        return carry

    lax.fori_loop(0, CPW // D, group_body, 0, unroll=False)
    for k in range(D):
        drain_out(k)


def kernel(input_tensor, item_size, emb_table):
    idx_w = input_tensor.reshape(NUM_ITEMS * MAX_SIZE).astype(jnp.int32)
    lens = item_size.astype(jnp.int32)

    mesh = plsc.VectorSubcoreMesh(core_axis_name="c", subcore_axis_name="s")
    run = functools.partial(
        pl.kernel,
        mesh=mesh,
        out_type=jax.ShapeDtypeStruct((NUM_ITEMS, EMB), jnp.float32),
        scratch_types=(
            [pltpu.VMEM((IPW,), jnp.int32),
             pltpu.VMEM((1, EMB), jnp.float32)]
            + [pltpu.VMEM((ROWS,), jnp.int32) for _ in range(D)]
            + [pltpu.VMEM((ROWS,), jnp.int32) for _ in range(D)]
            + [pltpu.VMEM((C, EMB), jnp.float32) for _ in range(D)]
            + [pltpu.VMEM((C, EMB), jnp.float32) for _ in range(D)]
            + [pltpu.SemaphoreType.DMA for _ in range(3 * D)]
        ),
    )(_sc_body)
    return run(idx_w, lens, emb_table)
